# baseline (speedup 1.0000x reference)
import functools

import jax
import jax.numpy as jnp
from jax import lax
from jax.experimental import pallas as pl
from jax.experimental.pallas import tpu as pltpu
from jax.experimental.pallas import tpu_sc as plsc

_NUM_HEADS = 8
_N_PER_HEAD = 100000
_TOTAL_N = _NUM_HEADS * _N_PER_HEAD
_D = 32
_B_ROWS = 16384
_B = _B_ROWS * _NUM_HEADS

_NC = 2
_NS = 16
_L = 16
_NW = _NC * _NS
_BPW = _B // _NW
_CH = 2048
_CHUNKS = _BPW // _CH


def _emb_body(ids_hbm, table_hbm, out_hbm, idx_v, rows_v, sem):
    wid = lax.axis_index("s") * _NC + lax.axis_index("c")
    off = (lax.iota(jnp.int32, _L) % jnp.int32(_NUM_HEADS)) * jnp.int32(_N_PER_HEAD)
    for c in range(_CHUNKS):
        base = wid * _BPW + c * _CH
        pltpu.sync_copy(ids_hbm.at[pl.ds(base, _CH)], idx_v)

        def _shift(j, carry):
            s = idx_v[pl.ds(j * _L, _L)] + off
            s = jnp.minimum(jnp.maximum(s, jnp.int32(0)), jnp.int32(_TOTAL_N - 1))
            idx_v[pl.ds(j * _L, _L)] = s
            return carry

        lax.fori_loop(0, _CH // _L, _shift, 0)
        pltpu.async_copy(table_hbm.at[idx_v], rows_v, sem).wait()
        pltpu.sync_copy(rows_v, out_hbm.at[pl.ds(base, _CH)])


@jax.jit
def kernel(input_ids, table):
    gather = functools.partial(
        pl.kernel,
        out_type=jax.ShapeDtypeStruct((_B, _D), jnp.float32),
        mesh=plsc.VectorSubcoreMesh(core_axis_name="c", subcore_axis_name="s"),
        scratch_types=[
            pltpu.VMEM((_CH,), jnp.int32),
            pltpu.VMEM((_CH, _D), jnp.float32),
            pltpu.SemaphoreType.DMA,
        ],
        compiler_params=pltpu.CompilerParams(use_tc_tiling_on_sc=False),
    )(_emb_body)
    out = gather(input_ids.reshape(_B), table)
    return out.reshape(_B_ROWS, _NUM_HEADS, _D)
